# Initial kernel scaffold; baseline (speedup 1.0000x reference)
import jax
import jax.numpy as jnp
from jax.experimental import pallas as pl


def kernel(x, xk1, xk3, xk6, adj, msk, W, b, Wd, bd):
    raise NotImplementedError("write your pallas kernel here")


# speedup vs baseline: 13.0006x; 13.0006x over previous
import functools

import jax
import jax.numpy as jnp
from jax import lax
from jax.experimental import pallas as pl
from jax.experimental.pallas import tpu as pltpu
from jax.experimental.pallas import tpu_sc as plsc

N = 10000
E = 320000
D = 128

NC = 2
NS = 16
K = 80
E_PER_TILE = E // NS
STEPS = E_PER_TILE // K
RC = K
NH = 10240
N_RCHUNKS = (N + RC - 1) // RC
RK_PER_TILE = (N_RCHUNKS + NS - 1) // NS

_f32 = jnp.float32


def _sc_body(x0, x1, x2, x3, src3, dst3,
             s0, s1, s2, s3, degh,
             acc,
             sidx, didxb0, didxb1, rows0, rows1, hist,
             sem0, sem1, isem0, isem1):
    cid = lax.axis_index("c")
    sid = lax.axis_index("s")

    pltpu.sync_copy(src3.at[pl.ds(sid * E_PER_TILE, E_PER_TILE)], sidx)

    def _fill_rows0(val):
        def body(i, carry):
            for j in range(D // 16):
                rows0[i, pl.ds(16 * j, 16)] = jnp.full((16,), carry, _f32)
            return carry

        lax.fori_loop(0, K, body, val)

    _fill_rows0(jnp.float32(0.0))

    def _for_chunks(fn):
        def body(k, carry):
            c = sid + NS * k

            @pl.when(c < N_RCHUNKS)
            def _():
                fn(pl.ds(c * RC, RC))

            return carry

        lax.fori_loop(0, RK_PER_TILE, body, 0)

    def _zero_acc(sl):
        pltpu.sync_copy(rows0, acc.at[sl])

    def _drain(buf, sem):
        pltpu.make_async_copy(x0.at[pl.ds(0, K)], buf, sem).wait()

    def _drain_idx(buf, sem):
        pltpu.make_async_copy(dst3.at[pl.ds(0, K)], buf, sem).wait()

    def _sidx(i):
        return sidx.at[pl.ds(i * K, K)]

    def _fire_idx(i, buf, sem):
        b = sid * E_PER_TILE + i * K
        pltpu.async_copy(dst3.at[pl.ds(b, K)], buf, sem)

    def _hist_chunk(buf):
        for v in range(K // 16):
            vec = buf[pl.ds(16 * v, 16)]
            cnt, m = plsc.scan_count(vec)
            plsc.addupdate_scatter(hist, [vec], cnt.astype(_f32), mask=m)

    def _edge_pass(xh, do_hist=False):
        _fire_idx(0, didxb0, isem0)
        _fire_idx(1, didxb1, isem1)
        pltpu.async_copy(xh.at[_sidx(0)], rows0, sem0)
        pltpu.async_copy(xh.at[_sidx(1)], rows1, sem1)

        def step(j, carry):
            i = 2 * j
            _drain(rows0, sem0)
            _drain_idx(didxb0, isem0)
            pltpu.sync_copy(rows0, acc.at[didxb0], add=True)
            if do_hist:
                @pl.when(cid == 0)
                def _():
                    _hist_chunk(didxb0)

            @pl.when(i + 2 < STEPS)
            def _():
                _fire_idx(i + 2, didxb0, isem0)
                pltpu.async_copy(xh.at[_sidx(i + 2)], rows0, sem0)

            _drain(rows1, sem1)
            _drain_idx(didxb1, isem1)
            pltpu.sync_copy(rows1, acc.at[didxb1], add=True)
            if do_hist:
                @pl.when(cid == 0)
                def _():
                    _hist_chunk(didxb1)

            @pl.when(i + 3 < STEPS)
            def _():
                _fire_idx(i + 3, didxb1, isem1)
                pltpu.async_copy(xh.at[_sidx(i + 3)], rows1, sem1)

            return carry

        lax.fori_loop(0, STEPS // 2, step, 0)

    def _zero_hist(i, carry):
        hist[pl.ds(16 * i, 16)] = jnp.zeros((16,), _f32)
        return carry

    lax.fori_loop(0, NH // 16, _zero_hist, 0)

    _for_chunks(_zero_acc)
    plsc.subcore_barrier()

    @pl.when(cid == 0)
    def _():
        _edge_pass(x0, do_hist=True)

    @pl.when(cid == 1)
    def _():
        _edge_pass(x2)

    @pl.when(cid == 0)
    def _():
        pltpu.sync_copy(hist, degh.at[sid])

    _fill_rows0(jnp.float32(0.0))
    plsc.subcore_barrier()

    def _write1(sl):
        @pl.when(cid == 0)
        def _():
            pltpu.sync_copy(acc.at[sl], s0.at[sl])

        @pl.when(cid == 1)
        def _():
            pltpu.sync_copy(acc.at[sl], s2.at[sl])

        pltpu.sync_copy(rows0, acc.at[sl])

    _for_chunks(_write1)
    plsc.subcore_barrier()

    @pl.when(cid == 0)
    def _():
        _edge_pass(x1)

    @pl.when(cid == 1)
    def _():
        _edge_pass(x3)

    plsc.subcore_barrier()

    def _write2(sl):
        @pl.when(cid == 0)
        def _():
            pltpu.sync_copy(acc.at[sl], s1.at[sl])

        @pl.when(cid == 1)
        def _():
            pltpu.sync_copy(acc.at[sl], s3.at[sl])

    _for_chunks(_write2)


@functools.cache
def _make_sc_aggregate():
  return pl.kernel(
    _sc_body,
    out_type=(
        jax.ShapeDtypeStruct((N, D), _f32),
        jax.ShapeDtypeStruct((N, D), _f32),
        jax.ShapeDtypeStruct((N, D), _f32),
        jax.ShapeDtypeStruct((N, D), _f32),
        jax.ShapeDtypeStruct((NS, NH), _f32),
    ),
    mesh=plsc.VectorSubcoreMesh(core_axis_name="c", subcore_axis_name="s"),
    compiler_params=pltpu.CompilerParams(needs_layout_passes=False),
    scratch_types=[
        pltpu.VMEM_SHARED((N, D), _f32),
        pltpu.VMEM((E_PER_TILE,), jnp.int32),
        pltpu.VMEM((K,), jnp.int32),
        pltpu.VMEM((K,), jnp.int32),
        pltpu.VMEM((K, D), _f32),
        pltpu.VMEM((K, D), _f32),
        pltpu.VMEM((NH,), _f32),
        pltpu.SemaphoreType.DMA,
        pltpu.SemaphoreType.DMA,
        pltpu.SemaphoreType.DMA,
        pltpu.SemaphoreType.DMA,
    ],
  )


R = 1000


def _deg_reduce_body(degh_ref, out_ref):
    ones = jnp.ones((NS, 1), _f32)
    out_ref[...] = jax.lax.dot_general(
        degh_ref[...], ones, (((0,), (0,)), ((), ())),
        preferred_element_type=_f32)


def _deg_reduce(degh):
    return pl.pallas_call(
        _deg_reduce_body,
        out_shape=jax.ShapeDtypeStruct((NH, 1), _f32),
    )(degh)


def _tc_body(s0, s1, s2, s3, degc, mskc, w, b2, wd, bd, out):
    deg = degc[...]
    recip = 1.0 / jnp.maximum(deg, 1.0)
    pos = jnp.where(deg > 0.0, 1.0, 0.0)
    wmat = w[...]
    brow = b2[...]

    def emb(s_ref):
        z = jnp.dot(s_ref[...] * recip, wmat, preferred_element_type=_f32) + brow
        return jnp.maximum(z, 0.0) * pos

    h = emb(s0)
    c1 = emb(s1)
    c2 = emb(s2)
    c3 = emb(s3)
    wdm = wd[...]
    m = mskc[:, 0:1]
    bd0 = bd[0]

    def score(c):
        t = jnp.dot(c, wdm, preferred_element_type=_f32)
        return jnp.sum(h * t, axis=1)

    logits = jnp.stack([score(c1), score(c2), score(c3)], axis=1)
    out[...] = (logits + bd0) * m


def _tc_finish(s0, s1, s2, s3, degc, mskc, w, b2, wd, bd):
    full = lambda i: (0, 0)
    return pl.pallas_call(
        _tc_body,
        grid=(N // R,),
        in_specs=[
            pl.BlockSpec((R, D), lambda i: (i, 0)),
            pl.BlockSpec((R, D), lambda i: (i, 0)),
            pl.BlockSpec((R, D), lambda i: (i, 0)),
            pl.BlockSpec((R, D), lambda i: (i, 0)),
            pl.BlockSpec((R, 1), lambda i: (i, 0)),
            pl.BlockSpec((R, 1), lambda i: (i, 0)),
            pl.BlockSpec((D, D), full),
            pl.BlockSpec((1, D), full),
            pl.BlockSpec((D, D), full),
            pl.BlockSpec(memory_space=pltpu.SMEM),
        ],
        out_specs=pl.BlockSpec((R, 3), lambda i: (i, 0)),
        out_shape=jax.ShapeDtypeStruct((N, 3), _f32),
    )(s0, s1, s2, s3, degc, mskc, w, b2, wd, bd)


def kernel(x, xk1, xk3, xk6, adj, msk, W, b, Wd, bd):
    adj32 = jnp.asarray(adj, jnp.int32)
    src3 = adj32[0]
    dst3 = adj32[1]
    s0, s1, s2, s3, degh = _make_sc_aggregate()(x, xk1, xk3, xk6,
                                                src3, dst3)
    degc = _deg_reduce(degh)[:N]
    out = _tc_finish(s0, s1, s2, s3, degc,
                     msk.reshape(N, 1), W, b.reshape(1, D), Wd, bd)
    return out.T
